# baseline (speedup 1.0000x reference)
import jax
import jax.numpy as jnp
from jax.experimental import pallas as pl
from jax.experimental.pallas import tpu as pltpu


TILE_N = 4096
M_CHUNKS = 4


def _augment_t(t, scale_xyz, sq_first):
    y = t * t
    s = (y[0:1, :] + y[1:2, :]) + y[2:3, :]
    hi = s.astype(jnp.bfloat16).astype(jnp.float32)
    r1 = s - hi
    mid = r1.astype(jnp.bfloat16).astype(jnp.float32)
    lo = r1 - mid
    ones = jnp.ones_like(s)
    parts = [scale_xyz * t]
    if sq_first:
        parts += [hi, mid, lo, ones, ones, ones]
    else:
        parts += [ones, ones, ones, hi, mid, lo]
    return jnp.concatenate(parts, axis=0)


def _chamfer_body(x1_ref, x2_ref, dist1_ref, dist2_ref, amat_ref, bmat_ref):
    i = pl.program_id(1)

    @pl.when(i == 0)
    def _build_b():
        b9 = _augment_t(jnp.transpose(x2_ref[0], (1, 0)), 1.0, False)
        bmat_ref[...] = jnp.transpose(b9, (1, 0)).astype(jnp.bfloat16)

    a9 = _augment_t(jnp.transpose(x1_ref[0], (1, 0)), -2.0, True)
    amat_ref[...] = jnp.transpose(a9, (1, 0)).astype(jnp.bfloat16)
    amat = amat_ref[...]
    m = bmat_ref.shape[0]
    mc = m // M_CHUNKS
    rowmin = None
    colmins = []
    for c in range(M_CHUNKS):
        d = jax.lax.dot_general(
            amat, bmat_ref[c * mc:(c + 1) * mc, :],
            (((1,), (1,)), ((), ())),
            preferred_element_type=jnp.float32,
        )
        rm = jnp.min(d, axis=1, keepdims=True)
        rowmin = rm if rowmin is None else jnp.minimum(rowmin, rm)
        colmins.append(jnp.min(d, axis=0, keepdims=True))
    dist1_ref[0, :, :] = jnp.transpose(rowmin, (1, 0))
    partial = jnp.concatenate(colmins, axis=1)

    @pl.when(i == 0)
    def _init():
        dist2_ref[0, :, :] = partial

    @pl.when(i > 0)
    def _acc():
        dist2_ref[0, :, :] = jnp.minimum(dist2_ref[0, :, :], partial)


@jax.jit
def kernel(input1, input2):
    b, n, _ = input1.shape
    m = input2.shape[1]
    grid = (b, n // TILE_N)
    dist1, dist2 = pl.pallas_call(
        _chamfer_body,
        grid=grid,
        in_specs=[
            pl.BlockSpec((1, TILE_N, 3), lambda bi, i: (bi, i, 0)),
            pl.BlockSpec((1, m, 3), lambda bi, i: (bi, 0, 0)),
        ],
        out_specs=[
            pl.BlockSpec((1, 1, TILE_N), lambda bi, i: (bi, 0, i)),
            pl.BlockSpec((1, 1, m), lambda bi, i: (bi, 0, 0)),
        ],
        out_shape=[
            jax.ShapeDtypeStruct((b, 1, n), jnp.float32),
            jax.ShapeDtypeStruct((b, 1, m), jnp.float32),
        ],
        scratch_shapes=[
            pltpu.VMEM((TILE_N, 9), jnp.bfloat16),
            pltpu.VMEM((m, 9), jnp.bfloat16),
        ],
    )(input1, input2)
    return dist1[:, 0, :], dist2[:, 0, :]


# speedup vs baseline: 1.5004x; 1.5004x over previous
import jax
import jax.numpy as jnp
from jax.experimental import pallas as pl
from jax.experimental.pallas import tpu as pltpu


TILE_N = 4096
M_CHUNKS = 4


def _augment_t(t, scale_xyz, sq_first):
    y = t * t
    s = (y[0:1, :] + y[1:2, :]) + y[2:3, :]
    hi = s.astype(jnp.bfloat16).astype(jnp.float32)
    r1 = s - hi
    mid = r1.astype(jnp.bfloat16).astype(jnp.float32)
    lo = r1 - mid
    ones = jnp.ones_like(s)
    parts = [scale_xyz * t]
    if sq_first:
        parts += [hi, mid, lo, ones, ones, ones]
    else:
        parts += [ones, ones, ones, hi, mid, lo]
    return jnp.concatenate(parts, axis=0)


def _chamfer_body(x1_ref, x2_ref, dist1_ref, dist2_ref, amat_ref, bmat_ref):
    i = pl.program_id(1)

    @pl.when(i == 0)
    def _build_b():
        b9 = _augment_t(x2_ref[0], 1.0, False)
        bmat_ref[...] = jnp.transpose(b9, (1, 0)).astype(jnp.bfloat16)

    a9 = _augment_t(x1_ref[0], -2.0, True)
    amat_ref[...] = jnp.transpose(a9, (1, 0)).astype(jnp.bfloat16)
    amat = amat_ref[...]
    m = bmat_ref.shape[0]
    mc = m // M_CHUNKS
    rowmin = None
    colmins = []
    for c in range(M_CHUNKS):
        d = jax.lax.dot_general(
            amat, bmat_ref[c * mc:(c + 1) * mc, :],
            (((1,), (1,)), ((), ())),
            preferred_element_type=jnp.float32,
        )
        rm = jnp.min(d, axis=1, keepdims=True)
        rowmin = rm if rowmin is None else jnp.minimum(rowmin, rm)
        colmins.append(jnp.min(d, axis=0, keepdims=True))
    dist1_ref[0, :, :] = jnp.transpose(rowmin, (1, 0))
    partial = jnp.concatenate(colmins, axis=1)

    @pl.when(i == 0)
    def _init():
        dist2_ref[0, :, :] = partial

    @pl.when(i > 0)
    def _acc():
        dist2_ref[0, :, :] = jnp.minimum(dist2_ref[0, :, :], partial)


@jax.jit
def kernel(input1, input2):
    b, n, _ = input1.shape
    m = input2.shape[1]
    x1t = jnp.transpose(input1, (0, 2, 1))
    x2t = jnp.transpose(input2, (0, 2, 1))
    grid = (b, n // TILE_N)
    dist1, dist2 = pl.pallas_call(
        _chamfer_body,
        grid=grid,
        in_specs=[
            pl.BlockSpec((1, 3, TILE_N), lambda bi, i: (bi, 0, i)),
            pl.BlockSpec((1, 3, m), lambda bi, i: (bi, 0, 0)),
        ],
        out_specs=[
            pl.BlockSpec((1, 1, TILE_N), lambda bi, i: (bi, 0, i)),
            pl.BlockSpec((1, 1, m), lambda bi, i: (bi, 0, 0)),
        ],
        out_shape=[
            jax.ShapeDtypeStruct((b, 1, n), jnp.float32),
            jax.ShapeDtypeStruct((b, 1, m), jnp.float32),
        ],
        scratch_shapes=[
            pltpu.VMEM((TILE_N, 9), jnp.bfloat16),
            pltpu.VMEM((m, 9), jnp.bfloat16),
        ],
    )(x1t, x2t)
    return dist1[:, 0, :], dist2[:, 0, :]
